# baseline (speedup 1.0000x reference)
import jax
import jax.numpy as jnp
from jax import lax
from jax.experimental import pallas as pl
from jax.experimental.pallas import tpu as pltpu
from jax.experimental.pallas import tpu_sc as plsc

TIME_T = 0.5
N_CORES = 2
N_SUBCORES = 16
LANES = 16
NW = N_CORES * N_SUBCORES

B = 8192
K = 16
D = 512
CHUNK = 4
B_PER_W = B // NW
N_CHUNKS = B_PER_W // CHUNK
DCH = D // LANES
MAXROWS = 80


def _body(tn_hbm, time_hbm, nodes_hbm, table_hbm, out_hbm,
          tn_v, time_v, nodes_v,
          idx_v0, idx_v1, wgt_v0, wgt_v1,
          nb_v0, nb_v1, out_v0, out_v1,
          seg_s0, seg_s1,
          sem_nb0, sem_nb1, sem_out0, sem_out1):
    wid = lax.axis_index("s") * N_CORES + lax.axis_index("c")
    base = wid * B_PER_W

    pltpu.sync_copy(tn_hbm.at[pl.ds(base * K, B_PER_W * K)], tn_v)
    pltpu.sync_copy(time_hbm.at[pl.ds(base * K, B_PER_W * K)], time_v)
    pltpu.sync_copy(nodes_hbm.at[pl.ds(base, B_PER_W)], nodes_v)

    idx_bufs = (idx_v0, idx_v1)
    wgt_bufs = (wgt_v0, wgt_v1)
    nb_bufs = (nb_v0, nb_v1)
    out_bufs = (out_v0, out_v1)
    seg_bufs = (seg_s0, seg_s1)
    nb_sems = (sem_nb0, sem_nb1)
    out_sems = (sem_out0, sem_out1)

    lane = lax.iota(jnp.int32, LANES)
    lane0 = lane == 0

    for q in range(0, MAXROWS, LANES):
        idx_v0[pl.ds(q, LANES)] = lane + q
        idx_v1[pl.ds(q, LANES)] = lane + q

    def issue_chunk(i, par):
        idxb, wgtb, seg = idx_bufs[par], wgt_bufs[par], seg_bufs[par]
        m_tot = jnp.int32(0)
        for l in range(CHUNK):
            n16 = i * (CHUNK * K) + l * K
            t = time_v[pl.ds(n16, K)]
            tn = tn_v[pl.ds(n16, K)]
            msk = t <= TIME_T
            w = jnp.where(msk, jnp.exp((t - TIME_T) * 0.01), 0.0)
            s = jnp.sum(w)
            tot = jnp.where(s == 0.0, 1.0, s)
            wn = w / jnp.broadcast_to(tot, (K,))
            rs_v = jnp.broadcast_to(jnp.sum(wn) + 1.0, (K,))
            fw = wn / rs_v
            inv_v = 1.0 / rs_v
            mi = msk.astype(jnp.int32)
            cnt = plsc.cumsum(mi)
            c = cnt[K - 1]
            pos = m_tot + cnt - mi
            plsc.store_scatter(idxb, [pos], tn, mask=msk)
            plsc.store_scatter(wgtb, [pos], fw, mask=msk)
            selfpos = jnp.broadcast_to(m_tot + c, (K,))
            selfid = plsc.load_gather(
                nodes_v, [jnp.broadcast_to(i * CHUNK + l, (K,))])
            plsc.store_scatter(idxb, [selfpos], selfid, mask=lane0)
            plsc.store_scatter(wgtb, [selfpos], inv_v, mask=lane0)
            seg[l] = m_tot
            m_tot = m_tot + c + 1
        seg[CHUNK] = m_tot
        n8 = lax.shift_right_logical(m_tot + 7, 3)
        seg[CHUNK + 1] = n8
        for nb8 in range(1, CHUNK * (K + 1) // 8 + 2):
            @pl.when(n8 == nb8)
            def _():
                pltpu.async_copy(
                    table_hbm.at[idxb.at[pl.ds(0, 8 * nb8)]],
                    nb_bufs[par].at[pl.ds(0, 8 * nb8)],
                    nb_sems[par])

    def wait_nb(par):
        n8 = seg_bufs[par][CHUNK + 1]
        for nb8 in range(1, CHUNK * (K + 1) // 8 + 2):
            @pl.when(n8 == nb8)
            def _():
                pltpu.make_async_copy(
                    table_hbm.at[idx_bufs[par].at[pl.ds(0, 8 * nb8)]],
                    nb_bufs[par].at[pl.ds(0, 8 * nb8)],
                    nb_sems[par]).wait()

    def out_slice(i):
        return out_hbm.at[pl.ds(base + i * CHUNK, CHUNK)]

    def wait_out(i, par):
        pltpu.make_async_copy(out_bufs[par], out_slice(i), out_sems[par]).wait()

    zero_v = jnp.zeros((LANES,), jnp.float32)
    HALF = DCH // 2

    def compute_chunk(par):
        nb, wgtb, ob, seg = nb_bufs[par], wgt_bufs[par], out_bufs[par], seg_bufs[par]
        for l in range(CHUNK):
            o = seg[l]
            e = seg[l + 1]
            half_terms = lax.shift_right_logical(e - o, 1)
            for h in range(2):
                def term(j, accs, h=h):
                    fwv = plsc.load_gather(wgtb, [jnp.broadcast_to(j, (K,))])
                    return tuple(
                        accs[d] + fwv * nb[j, pl.ds((h * HALF + d) * LANES,
                                                    LANES)]
                        for d in range(HALF))

                def fma2(p, accs, h=h):
                    j = o + 2 * p
                    return term(j + 1, term(j, accs, h=h), h=h)

                accs = lax.fori_loop(0, half_terms, fma2,
                                     tuple(zero_v for _ in range(HALF)))
                accs = lax.cond(
                    ((e - o) & 1) == 1, lambda a, h=h: term(e - 1, a, h=h),
                    lambda a: a, accs)
                for d in range(HALF):
                    ob[l, pl.ds((h * HALF + d) * LANES, LANES)] = accs[d]

    issue_chunk(0, 0)
    issue_chunk(1, 1)

    def outer(j, _):
        for par in range(2):
            i = 2 * j + par
            wait_nb(par)

            @pl.when(i >= 2)
            def _():
                wait_out(i - 2, par)

            compute_chunk(par)

            @pl.when(i + 2 < N_CHUNKS)
            def _():
                issue_chunk(i + 2, par)

            pltpu.async_copy(out_bufs[par], out_slice(i), out_sems[par])
        return 0

    lax.fori_loop(0, N_CHUNKS // 2, outer, 0)
    wait_out(N_CHUNKS - 2, 0)
    wait_out(N_CHUNKS - 1, 1)


@jax.jit
def _run(nodes, tn_flat, time_flat, table):
    mesh = plsc.VectorSubcoreMesh(
        core_axis_name="c", subcore_axis_name="s",
        num_cores=N_CORES, num_subcores=N_SUBCORES)
    f = pl.kernel(
        _body,
        out_type=jax.ShapeDtypeStruct((B, D), jnp.float32),
        mesh=mesh,
        compiler_params=pltpu.CompilerParams(needs_layout_passes=False),
        scratch_types=[
            pltpu.VMEM((B_PER_W * K,), jnp.int32),
            pltpu.VMEM((B_PER_W * K,), jnp.float32),
            pltpu.VMEM((B_PER_W,), jnp.int32),
            pltpu.VMEM((MAXROWS,), jnp.int32),
            pltpu.VMEM((MAXROWS,), jnp.int32),
            pltpu.VMEM((MAXROWS,), jnp.float32),
            pltpu.VMEM((MAXROWS,), jnp.float32),
            pltpu.VMEM((MAXROWS, D), jnp.float32),
            pltpu.VMEM((MAXROWS, D), jnp.float32),
            pltpu.VMEM((CHUNK, D), jnp.float32),
            pltpu.VMEM((CHUNK, D), jnp.float32),
            pltpu.SMEM((8,), jnp.int32),
            pltpu.SMEM((8,), jnp.int32),
            pltpu.SemaphoreType.DMA,
            pltpu.SemaphoreType.DMA,
            pltpu.SemaphoreType.DMA,
            pltpu.SemaphoreType.DMA,
        ],
    )
    return f(tn_flat, time_flat, nodes, table)


def kernel(nodes, to_neighs, time_neighs, features_table):
    tn_flat = to_neighs.reshape(-1)
    time_flat = time_neighs.reshape(-1)
    return _run(nodes, tn_flat, time_flat, features_table)


# speedup vs baseline: 1.2863x; 1.2863x over previous
import jax
import jax.numpy as jnp
from jax import lax
from jax.experimental import pallas as pl
from jax.experimental.pallas import tpu as pltpu
from jax.experimental.pallas import tpu_sc as plsc

TIME_T = 0.5
N_CORES = 2
N_SUBCORES = 16
LANES = 16
NW = N_CORES * N_SUBCORES

B = 8192
K = 16
D = 512
CHUNK = 4
B_PER_W = B // NW
N_CHUNKS = B_PER_W // CHUNK
DCH = D // LANES
MAXROWS = 80


def _body(tn_hbm, time_hbm, nodes_hbm, table_hbm, out_hbm,
          tn_v, time_v, nodes_v,
          idx_v0, idx_v1, wgt_v0, wgt_v1,
          nb_v0, nb_v1, out_v0, out_v1,
          seg_s0, seg_s1,
          sem_nb0, sem_nb1, sem_out0, sem_out1):
    wid = lax.axis_index("s") * N_CORES + lax.axis_index("c")
    base = wid * B_PER_W

    pltpu.sync_copy(tn_hbm.at[pl.ds(base * K, B_PER_W * K)], tn_v)
    pltpu.sync_copy(time_hbm.at[pl.ds(base * K, B_PER_W * K)], time_v)
    pltpu.sync_copy(nodes_hbm.at[pl.ds(base, B_PER_W)], nodes_v)

    idx_bufs = (idx_v0, idx_v1)
    wgt_bufs = (wgt_v0, wgt_v1)
    nb_bufs = (nb_v0, nb_v1)
    out_bufs = (out_v0, out_v1)
    seg_bufs = (seg_s0, seg_s1)
    nb_sems = (sem_nb0, sem_nb1)
    out_sems = (sem_out0, sem_out1)

    lane = lax.iota(jnp.int32, LANES)
    lane0 = lane == 0

    for q in range(0, MAXROWS, LANES):
        idx_v0[pl.ds(q, LANES)] = lane + q
        idx_v1[pl.ds(q, LANES)] = lane + q

    def issue_chunk(i, par):
        idxb, wgtb, seg = idx_bufs[par], wgt_bufs[par], seg_bufs[par]
        m_tot = jnp.int32(0)
        for l in range(CHUNK):
            n16 = i * (CHUNK * K) + l * K
            t = time_v[pl.ds(n16, K)]
            tn = tn_v[pl.ds(n16, K)]
            msk = t <= TIME_T
            w = jnp.where(msk, jnp.exp((t - TIME_T) * 0.01), 0.0)
            s = jnp.sum(w)
            tot = jnp.where(s == 0.0, 1.0, s)
            wn = w / jnp.broadcast_to(tot, (K,))
            rs_v = jnp.broadcast_to(jnp.sum(wn) + 1.0, (K,))
            fw = wn / rs_v
            inv_v = 1.0 / rs_v
            mi = msk.astype(jnp.int32)
            cnt = plsc.cumsum(mi)
            c = cnt[K - 1]
            pos = m_tot + cnt - mi
            plsc.store_scatter(idxb, [pos], tn, mask=msk)
            plsc.store_scatter(wgtb, [pos], fw, mask=msk)
            selfpos = jnp.broadcast_to(m_tot + c, (K,))
            selfid = plsc.load_gather(
                nodes_v, [jnp.broadcast_to(i * CHUNK + l, (K,))])
            plsc.store_scatter(idxb, [selfpos], selfid, mask=lane0)
            plsc.store_scatter(wgtb, [selfpos], inv_v, mask=lane0)
            seg[l] = m_tot
            m_tot = m_tot + c + 1
        seg[CHUNK] = m_tot
        n8 = lax.shift_right_logical(m_tot + 7, 3)
        seg[CHUNK + 1] = n8
        for nb8 in range(1, CHUNK * (K + 1) // 8 + 2):
            @pl.when(n8 == nb8)
            def _():
                pltpu.async_copy(
                    table_hbm.at[idxb.at[pl.ds(0, 8 * nb8)]],
                    nb_bufs[par].at[pl.ds(0, 8 * nb8)],
                    nb_sems[par])

    def wait_nb(par):
        n8 = seg_bufs[par][CHUNK + 1]
        for nb8 in range(1, CHUNK * (K + 1) // 8 + 2):
            @pl.when(n8 == nb8)
            def _():
                pltpu.make_async_copy(
                    table_hbm.at[idx_bufs[par].at[pl.ds(0, 8 * nb8)]],
                    nb_bufs[par].at[pl.ds(0, 8 * nb8)],
                    nb_sems[par]).wait()

    def out_slice(i):
        return out_hbm.at[pl.ds(base + i * CHUNK, CHUNK)]

    def wait_out(i, par):
        pltpu.make_async_copy(out_bufs[par], out_slice(i), out_sems[par]).wait()

    zero_v = jnp.zeros((LANES,), jnp.float32)
    HALF = DCH // 2

    def compute_chunk(par):
        nb, wgtb, ob, seg = nb_bufs[par], wgt_bufs[par], out_bufs[par], seg_bufs[par]
        for l in range(CHUNK):
            o = seg[l]
            e = seg[l + 1]
            for h in range(2):
                def fma(j, accs, h=h):
                    fwv = plsc.load_gather(wgtb, [jnp.broadcast_to(j, (K,))])
                    return tuple(
                        accs[d] + fwv * nb[j, pl.ds((h * HALF + d) * LANES,
                                                    LANES)]
                        for d in range(HALF))

                accs = lax.fori_loop(o, e, fma,
                                     tuple(zero_v for _ in range(HALF)))
                for d in range(HALF):
                    ob[l, pl.ds((h * HALF + d) * LANES, LANES)] = accs[d]

    issue_chunk(0, 0)
    issue_chunk(1, 1)

    def outer(j, _):
        for par in range(2):
            i = 2 * j + par
            wait_nb(par)

            @pl.when(i >= 2)
            def _():
                wait_out(i - 2, par)

            compute_chunk(par)

            @pl.when(i + 2 < N_CHUNKS)
            def _():
                issue_chunk(i + 2, par)

            pltpu.async_copy(out_bufs[par], out_slice(i), out_sems[par])
        return 0

    lax.fori_loop(0, N_CHUNKS // 2, outer, 0)
    wait_out(N_CHUNKS - 2, 0)
    wait_out(N_CHUNKS - 1, 1)


@jax.jit
def _run(nodes, tn_flat, time_flat, table):
    mesh = plsc.VectorSubcoreMesh(
        core_axis_name="c", subcore_axis_name="s",
        num_cores=N_CORES, num_subcores=N_SUBCORES)
    f = pl.kernel(
        _body,
        out_type=jax.ShapeDtypeStruct((B, D), jnp.float32),
        mesh=mesh,
        compiler_params=pltpu.CompilerParams(needs_layout_passes=False),
        scratch_types=[
            pltpu.VMEM((B_PER_W * K,), jnp.int32),
            pltpu.VMEM((B_PER_W * K,), jnp.float32),
            pltpu.VMEM((B_PER_W,), jnp.int32),
            pltpu.VMEM((MAXROWS,), jnp.int32),
            pltpu.VMEM((MAXROWS,), jnp.int32),
            pltpu.VMEM((MAXROWS,), jnp.float32),
            pltpu.VMEM((MAXROWS,), jnp.float32),
            pltpu.VMEM((MAXROWS, D), jnp.float32),
            pltpu.VMEM((MAXROWS, D), jnp.float32),
            pltpu.VMEM((CHUNK, D), jnp.float32),
            pltpu.VMEM((CHUNK, D), jnp.float32),
            pltpu.SMEM((8,), jnp.int32),
            pltpu.SMEM((8,), jnp.int32),
            pltpu.SemaphoreType.DMA,
            pltpu.SemaphoreType.DMA,
            pltpu.SemaphoreType.DMA,
            pltpu.SemaphoreType.DMA,
        ],
    )
    return f(tn_flat, time_flat, nodes, table)


def kernel(nodes, to_neighs, time_neighs, features_table):
    tn_flat = to_neighs.reshape(-1)
    time_flat = time_neighs.reshape(-1)
    return _run(nodes, tn_flat, time_flat, features_table)
